# Initial kernel scaffold; baseline (speedup 1.0000x reference)
import jax
import jax.numpy as jnp
from jax.experimental import pallas as pl


def kernel(scores, table):
    raise NotImplementedError("write your pallas kernel here")


# speedup vs baseline: 4.6535x; 4.6535x over previous
import jax
import jax.numpy as jnp
from jax import lax
from jax.experimental import pallas as pl
from jax.experimental.pallas import tpu as pltpu
from jax.experimental.pallas import tpu_sc as plsc

_NUM_HISTOGRAM = 1000
_EMBED = 16
_LOWER = 0.0
_STEP = (1.0 - 0.0) / _NUM_HISTOGRAM

_B, _L = 16384, 200
_N = _B * _L
_NC, _NS = 2, 16
_NW = _NC * _NS
_PER_W = _N // _NW
_CHUNK = 2048
_NCHUNK = _PER_W // _CHUNK
_LANES = 16
_GROUPS = _CHUNK // _LANES


def _body(scores_hbm, table_hbm, out_hbm, s_v, table_v, rows_v, sem):
    pltpu.sync_copy(table_hbm, table_v)
    wid = lax.axis_index("s") * _NC + lax.axis_index("c")
    base = wid * _PER_W
    iota16 = lax.iota(jnp.int32, _LANES)
    sidx_base = iota16 * _EMBED

    def chunk_body(ci, carry):
        off = base + ci * _CHUNK
        pltpu.sync_copy(scores_hbm.at[pl.ds(off, _CHUNK)], s_v)

        def group_body(g, c):
            s = s_v[pl.ds(g * _LANES, _LANES)]
            gidx = ((s - _LOWER) / _STEP).astype(jnp.int32) * _EMBED
            sidx = sidx_base + g * (_LANES * _EMBED)
            for col in range(_EMBED):
                vals = plsc.load_gather(table_v, [gidx])
                plsc.store_scatter(rows_v, [sidx], vals)
                if col + 1 < _EMBED:
                    gidx = gidx + 1
                    sidx = sidx + 1
            return c

        lax.fori_loop(0, _GROUPS, group_body, 0)
        pltpu.sync_copy(rows_v, out_hbm.at[pl.ds(off * _EMBED, _CHUNK * _EMBED)])
        return carry

    lax.fori_loop(0, _NCHUNK, chunk_body, 0)


def kernel(scores, table):
    f = pl.kernel(
        _body,
        out_type=jax.ShapeDtypeStruct((_N * _EMBED,), jnp.float32),
        mesh=plsc.VectorSubcoreMesh(core_axis_name="c", subcore_axis_name="s"),
        compiler_params=pltpu.CompilerParams(needs_layout_passes=False),
        scratch_types=[
            pltpu.VMEM((_CHUNK,), jnp.float32),
            pltpu.VMEM((_NUM_HISTOGRAM * _EMBED,), jnp.float32),
            pltpu.VMEM((_CHUNK * _EMBED,), jnp.float32),
            pltpu.SemaphoreType.DMA,
        ],
    )
    out = f(scores.reshape(_N), table.reshape(_NUM_HISTOGRAM * _EMBED))
    return out.reshape(_B, _L, _EMBED)
